# baseline (speedup 1.0000x reference)
import jax
import jax.numpy as jnp
from jax.experimental import pallas as pl
from jax.experimental.pallas import tpu as pltpu

_N_CHUNKS = 8


def _copy_body(t_hbm, o_hbm, *scratch):
    bufs = scratch[:_N_CHUNKS]
    isem, osem = scratch[_N_CHUNKS:]
    rows = o_hbm.shape[0]
    chunk = rows // _N_CHUNKS

    def in_copy(c):
        return pltpu.make_async_copy(
            t_hbm.at[pl.ds(c * chunk, chunk), :], bufs[c], isem.at[c])

    def out_copy(c):
        return pltpu.make_async_copy(
            bufs[c], o_hbm.at[pl.ds(c * chunk, chunk), :], osem.at[c])

    for c in range(_N_CHUNKS):
        in_copy(c).start()
    for c in range(_N_CHUNKS):
        in_copy(c).wait()
        out_copy(c).start()
    for c in range(_N_CHUNKS):
        out_copy(c).wait()


def kernel(x, table):
    T = x.shape[1]
    _, d = table.shape
    chunk = T // _N_CHUNKS
    return pl.pallas_call(
        _copy_body,
        in_specs=[pl.BlockSpec(memory_space=pltpu.MemorySpace.HBM)],
        out_specs=pl.BlockSpec(memory_space=pltpu.MemorySpace.HBM),
        scratch_shapes=(
            [pltpu.VMEM((chunk, d), table.dtype) for _ in range(_N_CHUNKS)]
            + [pltpu.SemaphoreType.DMA((_N_CHUNKS,)),
               pltpu.SemaphoreType.DMA((_N_CHUNKS,))]
        ),
        out_shape=jax.ShapeDtypeStruct((T, d), table.dtype),
        compiler_params=pltpu.CompilerParams(
            vmem_limit_bytes=60 * 1024 * 1024),
    )(table)
